# baseline (speedup 1.0000x reference)
import jax
import jax.numpy as jnp
from jax.experimental import pallas as pl
from jax.experimental.pallas import tpu as pltpu

_DEPTH = 384
_RT = 1024


def _body(x_ref, w_ref, bias_ref, basis_ref, out_ref, map_ref,
          max_ref, den_ref, found_ref):
    p = pl.program_id(0)
    r = pl.program_id(1)
    b = pl.program_id(2)
    base = r * _RT
    num_caps = x_ref.shape[2]

    x = x_ref[0, b, pl.ds(base, _RT)]
    feat = jnp.dot(x, w_ref[...], preferred_element_type=jnp.float32)
    feat = feat + bias_ref[...]
    m = feat * basis_ref[...]

    @pl.when(p == 0)
    def _():
        map_ref[0] = m
        tmax = jnp.max(m, axis=0, keepdims=True)
        tsum = jnp.sum(jnp.exp(m - tmax), axis=0, keepdims=True)

        @pl.when(r == 0)
        def _():
            max_ref[pl.ds(b, 1), :] = tmax
            den_ref[pl.ds(b, 1), :] = tsum

        @pl.when(r > 0)
        def _():
            old_max = max_ref[pl.ds(b, 1), :]
            new_max = jnp.maximum(old_max, tmax)
            den_ref[pl.ds(b, 1), :] = (den_ref[pl.ds(b, 1), :]
                                       * jnp.exp(old_max - new_max)
                                       + tsum * jnp.exp(tmax - new_max))
            max_ref[pl.ds(b, 1), :] = new_max

    @pl.when(p == 1)
    def _():
        mx = max_ref[pl.ds(b, 1), :]
        den = den_ref[pl.ds(b, 1), :]
        a = jnp.exp(m - mx) / den
        amax = 1.0 / den

        tie = a == amax
        tie_f = tie.astype(jnp.float32)
        rows = jax.lax.broadcasted_iota(jnp.int32, m.shape, 0)
        first = jnp.argmax(tie_f, axis=0).astype(jnp.int32).reshape(1, -1)
        have = jnp.max(tie_f, axis=0, keepdims=True)

        prev = jnp.where(r > 0, found_ref[pl.ds(b, 1), :], 0)
        own = (have > 0.0) & (prev == 0)
        found_ref[pl.ds(b, 1), :] = prev | have.astype(jnp.int32)
        sel = tie & (rows == first) & own

        sel_paired = jnp.concatenate(
            [sel[:, :_DEPTH], sel[:, :_DEPTH], sel[:, 2 * _DEPTH:]], axis=1)

        av = a * feat
        cols = jax.lax.broadcasted_iota(jnp.int32, m.shape, 1)
        out_ref[0] = jnp.where(cols >= 2 * _DEPTH, av * (1.0 / num_caps),
                               jnp.where(sel_paired, av, 0.0))


def kernel(feat_list, W, b, caps_basis):
    L, Bv, Nv = feat_list.shape[0], feat_list.shape[1], feat_list.shape[2]
    cin = feat_list.shape[-1]
    num_caps = caps_basis.shape[1]
    cout = caps_basis.shape[3]
    xs = feat_list.reshape(L, Bv, Nv * Nv, cin)
    slab = L - 2
    basis = caps_basis.reshape(num_caps, cout)
    bias2 = b.reshape(1, cout)
    n_r = num_caps // _RT
    f32 = jnp.float32

    out, attn_map = pl.pallas_call(
        _body,
        grid=(2, n_r, Bv),
        in_specs=[
            pl.BlockSpec((1, Bv, num_caps, cin),
                         lambda p, r, bb: (slab, 0, 0, 0)),
            pl.BlockSpec((cin, cout), lambda p, r, bb: (0, 0)),
            pl.BlockSpec((1, cout), lambda p, r, bb: (0, 0)),
            pl.BlockSpec((_RT, cout), lambda p, r, bb: (r, 0)),
        ],
        out_specs=(
            pl.BlockSpec((1, _RT, cout),
                         lambda p, r, bb: (jnp.where(p == 1, bb, 0),
                                           jnp.where(p == 1, r, 0), 0)),
            pl.BlockSpec((1, _RT, cout),
                         lambda p, r, bb, _B=Bv - 1, _R=n_r - 1:
                         (jnp.where(p == 0, bb, _B),
                          jnp.where(p == 0, r, _R), 0)),
        ),
        out_shape=(
            jax.ShapeDtypeStruct((Bv, num_caps, cout), f32),
            jax.ShapeDtypeStruct((Bv, num_caps, cout), f32),
        ),
        scratch_shapes=[
            pltpu.VMEM((Bv, cout), f32),
            pltpu.VMEM((Bv, cout), f32),
            pltpu.VMEM((Bv, cout), jnp.int32),
        ],
        compiler_params=pltpu.CompilerParams(
            dimension_semantics=("arbitrary", "arbitrary", "arbitrary"),
        ),
    )(xs, W, bias2, basis)
    return (out, attn_map)


# speedup vs baseline: 1.9001x; 1.9001x over previous
import jax
import jax.numpy as jnp
from jax.experimental import pallas as pl
from jax.experimental.pallas import tpu as pltpu

_DEPTH = 384
_CT = 512


def _body(x_ref, w_ref, bias_ref, basis_ref, out_ref, map_ref,
          max_ref, den_ref, found_ref):
    p = pl.program_id(0)
    r = pl.program_id(1)
    b = pl.program_id(2)
    base = r * _CT
    num_caps = x_ref.shape[2]

    x = x_ref[0, b, pl.ds(base, _CT)]
    featT = jax.lax.dot_general(w_ref[...], x, (((0,), (1,)), ((), ())),
                                preferred_element_type=jnp.float32)
    featT = featT + bias_ref[...]
    mT = featT * basis_ref[...]

    @pl.when(p == 0)
    def _():
        map_ref[0] = mT
        tmax = jnp.max(mT, axis=1, keepdims=True)
        tsum = jnp.sum(jnp.exp(mT - tmax), axis=1, keepdims=True)

        @pl.when(r == 0)
        def _():
            max_ref[b] = tmax
            den_ref[b] = tsum

        @pl.when(r > 0)
        def _():
            old_max = max_ref[b]
            new_max = jnp.maximum(old_max, tmax)
            den_ref[b] = (den_ref[b] * jnp.exp(old_max - new_max)
                          + tsum * jnp.exp(tmax - new_max))
            max_ref[b] = new_max

    @pl.when(p == 1)
    def _():
        mx = max_ref[b]
        den = den_ref[b]
        a = jnp.exp(mT - mx) / den
        amax = 1.0 / den

        tie = a == amax
        tie_f = tie.astype(jnp.float32)
        caps_i = jax.lax.broadcasted_iota(jnp.int32, mT.shape, 1)
        first = jnp.argmax(tie_f, axis=1).astype(jnp.int32).reshape(-1, 1)
        have = jnp.max(tie_f, axis=1, keepdims=True)

        prev = jnp.where(r > 0, found_ref[b], 0)
        own = (have > 0.0) & (prev == 0)
        found_ref[b] = prev | have.astype(jnp.int32)
        sel = tie & (caps_i == first) & own

        sel_paired = jnp.concatenate(
            [sel[:_DEPTH], sel[:_DEPTH], sel[2 * _DEPTH:]], axis=0)

        av = a * featT
        col_i = jax.lax.broadcasted_iota(jnp.int32, mT.shape, 0)
        out_ref[0] = jnp.where(col_i >= 2 * _DEPTH, av * (1.0 / num_caps),
                               jnp.where(sel_paired, av, 0.0))


def kernel(feat_list, W, b, caps_basis):
    L, Bv, Nv = feat_list.shape[0], feat_list.shape[1], feat_list.shape[2]
    cin = feat_list.shape[-1]
    num_caps = caps_basis.shape[1]
    cout = caps_basis.shape[3]
    xs = feat_list.reshape(L, Bv, Nv * Nv, cin)
    slab = L - 2
    basisT = jnp.swapaxes(caps_basis.reshape(num_caps, cout), 0, 1)
    biasT = b.reshape(cout, 1)
    n_r = num_caps // _CT
    f32 = jnp.float32

    outT, mapT = pl.pallas_call(
        _body,
        grid=(2, n_r, Bv),
        in_specs=[
            pl.BlockSpec((1, Bv, num_caps, cin),
                         lambda p, r, bb: (slab, 0, 0, 0)),
            pl.BlockSpec((cin, cout), lambda p, r, bb: (0, 0)),
            pl.BlockSpec((cout, 1), lambda p, r, bb: (0, 0)),
            pl.BlockSpec((cout, _CT), lambda p, r, bb: (0, r)),
        ],
        out_specs=(
            pl.BlockSpec((1, cout, _CT),
                         lambda p, r, bb: (jnp.where(p == 1, bb, 0), 0,
                                           jnp.where(p == 1, r, 0))),
            pl.BlockSpec((1, cout, _CT),
                         lambda p, r, bb, _B=Bv - 1, _R=n_r - 1:
                         (jnp.where(p == 0, bb, _B), 0,
                          jnp.where(p == 0, r, _R))),
        ),
        out_shape=(
            jax.ShapeDtypeStruct((Bv, cout, num_caps), f32),
            jax.ShapeDtypeStruct((Bv, cout, num_caps), f32),
        ),
        scratch_shapes=[
            pltpu.VMEM((Bv, cout, 1), f32),
            pltpu.VMEM((Bv, cout, 1), f32),
            pltpu.VMEM((Bv, cout, 1), jnp.int32),
        ],
        compiler_params=pltpu.CompilerParams(
            dimension_semantics=("arbitrary", "arbitrary", "arbitrary"),
        ),
    )(xs, W, biasT, basisT)
    return (jnp.swapaxes(outT, 1, 2), jnp.swapaxes(mapT, 1, 2))


# speedup vs baseline: 2.8971x; 1.5247x over previous
import jax
import jax.numpy as jnp
from jax.experimental import pallas as pl
from jax.experimental.pallas import tpu as pltpu

_DEPTH = 384
_TILE = 96


def _body(x_ref, wt_ref, bias_ref, basis_ref, out_ref, map_ref, am_ref):
    c = pl.program_id(0)
    b = pl.program_id(1)
    num_caps = x_ref.shape[2]
    n_own = _DEPTH // _TILE

    x = x_ref[0, b]
    featT = jax.lax.dot_general(wt_ref[...], x, (((1,), (1,)), ((), ())),
                                preferred_element_type=jnp.float32)
    featT = featT + bias_ref[...]
    mT = featT * basis_ref[...]
    map_ref[0] = mT

    tmax = jnp.max(mT, axis=1, keepdims=True)
    e = jnp.exp(mT - tmax)
    den = jnp.sum(e, axis=1, keepdims=True)
    a = e / den
    amax = 1.0 / den

    tie_f = (a == amax).astype(jnp.float32)
    first = jnp.argmax(tie_f, axis=1).astype(jnp.int32).reshape(-1, 1)

    @pl.when(c < n_own)
    def _():
        am_ref[b, pl.ds(c * _TILE, _TILE)] = first

    pair_start = jnp.clip(c - n_own, 0, n_own - 1) * _TILE
    stored = am_ref[b, pl.ds(pair_start, _TILE)]
    sel_row = jnp.where(c < n_own, first, stored)

    caps_i = jax.lax.broadcasted_iota(jnp.int32, mT.shape, 1)
    av = a * featT
    routed = jnp.where(caps_i == sel_row, av, 0.0)
    out_ref[0] = jnp.where(c >= 2 * n_own, av * (1.0 / num_caps), routed)


def kernel(feat_list, W, b, caps_basis):
    L, Bv, Nv = feat_list.shape[0], feat_list.shape[1], feat_list.shape[2]
    cin = feat_list.shape[-1]
    num_caps = caps_basis.shape[1]
    cout = caps_basis.shape[3]
    xs = feat_list.reshape(L, Bv, Nv * Nv, cin)
    slab = L - 2
    basisT = jnp.swapaxes(caps_basis.reshape(num_caps, cout), 0, 1)
    wT = jnp.swapaxes(W, 0, 1)
    biasT = b.reshape(cout, 1)
    n_c = cout // _TILE
    f32 = jnp.float32

    outT, mapT = pl.pallas_call(
        _body,
        grid=(n_c, Bv),
        in_specs=[
            pl.BlockSpec((1, Bv, num_caps, cin),
                         lambda c, bb: (slab, 0, 0, 0)),
            pl.BlockSpec((_TILE, cin), lambda c, bb: (c, 0)),
            pl.BlockSpec((_TILE, 1), lambda c, bb: (c, 0)),
            pl.BlockSpec((_TILE, num_caps), lambda c, bb: (c, 0)),
        ],
        out_specs=(
            pl.BlockSpec((1, _TILE, num_caps), lambda c, bb: (bb, c, 0)),
            pl.BlockSpec((1, _TILE, num_caps), lambda c, bb: (bb, c, 0)),
        ),
        out_shape=(
            jax.ShapeDtypeStruct((Bv, cout, num_caps), f32),
            jax.ShapeDtypeStruct((Bv, cout, num_caps), f32),
        ),
        scratch_shapes=[pltpu.VMEM((Bv, _DEPTH, 1), jnp.int32)],
        compiler_params=pltpu.CompilerParams(
            dimension_semantics=("arbitrary", "arbitrary"),
        ),
    )(xs, wT, biasT, basisT)
    return (jnp.swapaxes(outT, 1, 2), jnp.swapaxes(mapT, 1, 2))


# speedup vs baseline: 2.9592x; 1.0215x over previous
import jax
import jax.numpy as jnp
from jax.experimental import pallas as pl
from jax.experimental.pallas import tpu as pltpu

_DEPTH = 384
_TILE = 96


def _body(x_ref, wt_ref, bias_ref, basis_ref, out_ref, map_ref, am_ref):
    c = pl.program_id(0)
    b = pl.program_id(1)
    num_caps = x_ref.shape[2]
    n_own = _DEPTH // _TILE

    x = x_ref[0, b]
    featT = jax.lax.dot_general(wt_ref[...], x, (((1,), (1,)), ((), ())),
                                preferred_element_type=jnp.float32)
    featT = featT + bias_ref[...]
    mT = featT * basis_ref[...]
    map_ref[0] = mT

    tmax = jnp.max(mT, axis=1, keepdims=True)
    e = jnp.exp(mT - tmax)
    den = jnp.sum(e, axis=1, keepdims=True)
    a = e / den
    amax = 1.0 / den

    tie = a == amax
    caps_i = jax.lax.broadcasted_iota(jnp.int32, mT.shape, 1)
    first = jnp.min(jnp.where(tie, caps_i, num_caps),
                    axis=1, keepdims=True)

    @pl.when(c < n_own)
    def _():
        am_ref[b, pl.ds(c * _TILE, _TILE)] = first

    pair_start = jnp.clip(c - n_own, 0, n_own - 1) * _TILE
    stored = am_ref[b, pl.ds(pair_start, _TILE)]
    sel_row = jnp.where(c < n_own, first, stored)

    av = a * featT
    routed = jnp.where(caps_i == sel_row, av, 0.0)
    out_ref[0] = jnp.where(c >= 2 * n_own, av * (1.0 / num_caps), routed)


def kernel(feat_list, W, b, caps_basis):
    L, Bv, Nv = feat_list.shape[0], feat_list.shape[1], feat_list.shape[2]
    cin = feat_list.shape[-1]
    num_caps = caps_basis.shape[1]
    cout = caps_basis.shape[3]
    xs = feat_list.reshape(L, Bv, Nv * Nv, cin)
    slab = L - 2
    basisT = jnp.swapaxes(caps_basis.reshape(num_caps, cout), 0, 1)
    wT = jnp.swapaxes(W, 0, 1)
    biasT = b.reshape(cout, 1)
    n_c = cout // _TILE
    f32 = jnp.float32

    outT, mapT = pl.pallas_call(
        _body,
        grid=(n_c, Bv),
        in_specs=[
            pl.BlockSpec((1, Bv, num_caps, cin),
                         lambda c, bb: (slab, 0, 0, 0)),
            pl.BlockSpec((_TILE, cin), lambda c, bb: (c, 0)),
            pl.BlockSpec((_TILE, 1), lambda c, bb: (c, 0)),
            pl.BlockSpec((_TILE, num_caps), lambda c, bb: (c, 0)),
        ],
        out_specs=(
            pl.BlockSpec((1, _TILE, num_caps), lambda c, bb: (bb, c, 0)),
            pl.BlockSpec((1, _TILE, num_caps), lambda c, bb: (bb, c, 0)),
        ),
        out_shape=(
            jax.ShapeDtypeStruct((Bv, cout, num_caps), f32),
            jax.ShapeDtypeStruct((Bv, cout, num_caps), f32),
        ),
        scratch_shapes=[pltpu.VMEM((Bv, _DEPTH, 1), jnp.int32)],
        compiler_params=pltpu.CompilerParams(
            dimension_semantics=("arbitrary", "arbitrary"),
        ),
    )(xs, wT, biasT, basisT)
    return (jnp.swapaxes(outT, 1, 2), jnp.swapaxes(mapT, 1, 2))


# speedup vs baseline: 3.7532x; 1.2683x over previous
import jax
import jax.numpy as jnp
from jax.experimental import pallas as pl
from jax.experimental.pallas import tpu as pltpu

_DEPTH = 384
_TILE = 216


def _body(x_ref, wt_ref, bias_ref, basis_ref, out_ref, map_ref, am_ref):
    c = pl.program_id(0)
    b = pl.program_id(1)
    num_caps = x_ref.shape[2]
    col0 = c * _TILE

    x = x_ref[0, b]
    featT = jax.lax.dot_general(wt_ref[...], x, (((1,), (1,)), ((), ())),
                                preferred_element_type=jnp.float32)
    featT = featT + bias_ref[...]
    mT = featT * basis_ref[...]
    map_ref[0] = mT

    tmax = jnp.max(mT, axis=1, keepdims=True)
    e = jnp.exp(mT - tmax)
    den = jnp.sum(e, axis=1, keepdims=True)
    amax = 1.0 / den
    ef = e * featT

    caps_i = jax.lax.broadcasted_iota(jnp.int32, mT.shape, 1)
    col_g = jax.lax.broadcasted_iota(jnp.int32, (_TILE, 1), 0) + col0
    stored = am_ref[b, pl.ds(col0, _TILE)]

    @pl.when(col0 < _DEPTH)
    def _():
        a = e / den
        tie = a == amax
        first = jnp.min(jnp.where(tie, caps_i, num_caps),
                        axis=1, keepdims=True)
        am_ref[b, pl.ds(col0 + _DEPTH, _TILE)] = first
        sel_row = jnp.where(col_g < _DEPTH, first, stored)
        sel = caps_i == sel_row
        out_ref[0] = jnp.where(sel, ef, 0.0) * amax

    @pl.when(col0 >= _DEPTH)
    def _():
        sel = caps_i == stored
        routed = jnp.where(sel, ef, 0.0)
        out_ref[0] = jnp.where(col_g >= 2 * _DEPTH, ef * (1.0 / num_caps),
                               routed) * amax


def kernel(feat_list, W, b, caps_basis):
    L, Bv, Nv = feat_list.shape[0], feat_list.shape[1], feat_list.shape[2]
    cin = feat_list.shape[-1]
    num_caps = caps_basis.shape[1]
    cout = caps_basis.shape[3]
    xs = feat_list.reshape(L, Bv, Nv * Nv, cin)
    slab = L - 2
    basisT = jnp.swapaxes(caps_basis.reshape(num_caps, cout), 0, 1)
    wT = jnp.swapaxes(W, 0, 1)
    biasT = b.reshape(cout, 1)
    n_c = cout // _TILE
    f32 = jnp.float32

    outT, mapT = pl.pallas_call(
        _body,
        grid=(n_c, Bv),
        in_specs=[
            pl.BlockSpec((1, Bv, num_caps, cin),
                         lambda c, bb: (slab, 0, 0, 0)),
            pl.BlockSpec((_TILE, cin), lambda c, bb: (c, 0)),
            pl.BlockSpec((_TILE, 1), lambda c, bb: (c, 0)),
            pl.BlockSpec((_TILE, num_caps), lambda c, bb: (c, 0)),
        ],
        out_specs=(
            pl.BlockSpec((1, _TILE, num_caps), lambda c, bb: (bb, c, 0)),
            pl.BlockSpec((1, _TILE, num_caps), lambda c, bb: (bb, c, 0)),
        ),
        out_shape=(
            jax.ShapeDtypeStruct((Bv, cout, num_caps), f32),
            jax.ShapeDtypeStruct((Bv, cout, num_caps), f32),
        ),
        scratch_shapes=[pltpu.VMEM((Bv, cout + _TILE, 1), jnp.int32)],
        compiler_params=pltpu.CompilerParams(
            dimension_semantics=("arbitrary", "arbitrary"),
        ),
    )(xs, wT, biasT, basisT)
    return (jnp.swapaxes(outT, 1, 2), jnp.swapaxes(mapT, 1, 2))
